# baseline (speedup 1.0000x reference)
import functools

import jax
import jax.numpy as jnp
from jax import lax
from jax.experimental import pallas as pl
from jax.experimental.pallas import tpu as pltpu
from jax.experimental.pallas import tpu_sc as plsc

N_TOKENS = 32768
DIM_IN = 4096
NUM_EXPERTS = 64
BT = 1024
HBT = BT // 2

NHALF = N_TOKENS // 2
NUM_WORKERS = 32
TOK_PER_W = NHALF // NUM_WORKERS
LANES = 16
GUNROLL = 4


def _gate_block(x0_ref, x1_ref, w_ref, b_ref, out_ref):
    lt0 = lax.dot_general(
        w_ref[...], x0_ref[...],
        dimension_numbers=(((1,), (1,)), ((), ())),
        preferred_element_type=jnp.float32,
    )
    lt1 = lax.dot_general(
        w_ref[...], x1_ref[...],
        dimension_numbers=(((1,), (1,)), ((), ())),
        preferred_element_type=jnp.float32,
    )
    logits_t = jnp.concatenate([lt0, lt1], axis=1) + b_ref[...]
    out_ref[...] = jnp.stack(
        [logits_t[:, j * TOK_PER_W:(j + 1) * TOK_PER_W]
         for j in range(BT // TOK_PER_W)], axis=0)


def _sc_top2(logits_hbm, v1_hbm, v2_hbm, i1_hbm, i2_hbm,
             lg_v, v1_v, v2_v, i1_v, i2_v):
    wid = lax.axis_index("s") * 2 + lax.axis_index("c")
    base = wid * TOK_PER_W
    pltpu.sync_copy(logits_hbm.at[wid], lg_v)

    def c_body(gq, carry):
        states = []
        for u in range(GUNROLL):
            m1 = jnp.full((LANES,), -jnp.inf, jnp.float32)
            m2 = jnp.full((LANES,), -jnp.inf, jnp.float32)
            i1 = jnp.zeros((LANES,), jnp.int32)
            i2 = jnp.zeros((LANES,), jnp.int32)
            states.append([m1, m2, i1, i2])
        for e in range(NUM_EXPERTS):
            e_s = jnp.full((LANES,), e, jnp.int32)
            for u in range(GUNROLL):
                m1, m2, i1, i2 = states[u]
                v = lg_v[e, pl.ds((gq * GUNROLL + u) * LANES, LANES)]
                gt1 = v > m1
                gt2 = v > m2
                m2 = jnp.where(gt1, m1, jnp.where(gt2, v, m2))
                i2 = jnp.where(gt1, i1, jnp.where(gt2, e_s, i2))
                m1 = jnp.where(gt1, v, m1)
                i1 = jnp.where(gt1, e_s, i1)
                states[u] = [m1, m2, i1, i2]
        for u in range(GUNROLL):
            m1, m2, i1, i2 = states[u]
            sl2 = pl.ds((gq * GUNROLL + u) * LANES, LANES)
            v1_v[sl2] = m1
            v2_v[sl2] = m2
            i1_v[sl2] = i1
            i2_v[sl2] = i2
        return carry

    lax.fori_loop(0, (TOK_PER_W // LANES) // GUNROLL, c_body, 0)
    pltpu.sync_copy(v1_v, v1_hbm.at[pl.ds(base, TOK_PER_W)])
    pltpu.sync_copy(v2_v, v2_hbm.at[pl.ds(base, TOK_PER_W)])
    pltpu.sync_copy(i1_v, i1_hbm.at[pl.ds(base, TOK_PER_W)])
    pltpu.sync_copy(i2_v, i2_hbm.at[pl.ds(base, TOK_PER_W)])


def _gate_half(x, W, b_col, half):
    nblk = NHALF // BT
    return pl.pallas_call(
        _gate_block,
        grid=(nblk,),
        in_specs=[
            pl.BlockSpec((HBT, DIM_IN), lambda i, h=half: (2 * (i + h * nblk), 0)),
            pl.BlockSpec((HBT, DIM_IN), lambda i, h=half: (2 * (i + h * nblk) + 1, 0)),
            pl.BlockSpec((NUM_EXPERTS, DIM_IN), lambda i: (0, 0)),
            pl.BlockSpec((NUM_EXPERTS, 1), lambda i: (0, 0)),
        ],
        out_specs=pl.BlockSpec((BT // TOK_PER_W, NUM_EXPERTS, TOK_PER_W),
                               lambda i: (i, 0, 0)),
        out_shape=jax.ShapeDtypeStruct((NUM_WORKERS, NUM_EXPERTS, TOK_PER_W),
                                       jnp.float32),
    )(x, x, W, b_col)


def _sc_half(logits):
    sc_call = functools.partial(
        pl.kernel,
        mesh=plsc.VectorSubcoreMesh(core_axis_name="c", subcore_axis_name="s"),
        out_type=[
            jax.ShapeDtypeStruct((NHALF,), jnp.float32),
            jax.ShapeDtypeStruct((NHALF,), jnp.float32),
            jax.ShapeDtypeStruct((NHALF,), jnp.int32),
            jax.ShapeDtypeStruct((NHALF,), jnp.int32),
        ],
        scratch_types=[
            pltpu.VMEM((NUM_EXPERTS, TOK_PER_W), jnp.float32),
            pltpu.VMEM((TOK_PER_W,), jnp.float32),
            pltpu.VMEM((TOK_PER_W,), jnp.float32),
            pltpu.VMEM((TOK_PER_W,), jnp.int32),
            pltpu.VMEM((TOK_PER_W,), jnp.int32),
        ],
    )(_sc_top2)
    return sc_call(logits)


@jax.jit
def kernel(x, W, b):
    b_col = b.reshape(NUM_EXPERTS, 1)
    logits_a = _gate_half(x, W, b_col, 0)
    logits_b = _gate_half(x, W, b_col, 1)
    v1a, v2a, i1a, i2a = _sc_half(logits_a)
    v1b, v2b, i1b, i2b = _sc_half(logits_b)
    vals = jnp.stack([jnp.concatenate([v1a, v1b]),
                      jnp.concatenate([v2a, v2b])], axis=1)
    idx = jnp.stack([jnp.concatenate([i1a, i1b]),
                     jnp.concatenate([i2a, i2b])], axis=1)
    return (vals, idx)


# speedup vs baseline: 1.0207x; 1.0207x over previous
import functools

import jax
import jax.numpy as jnp
from jax import lax
from jax.experimental import pallas as pl
from jax.experimental.pallas import tpu as pltpu
from jax.experimental.pallas import tpu_sc as plsc

N_TOKENS = 32768
DIM_IN = 4096
NUM_EXPERTS = 64
BT = 1024
HBT = BT // 2

NUM_WORKERS = 32
TOK_PER_W = N_TOKENS // NUM_WORKERS
LANES = 16
NCHUNK = 4
CTOK = TOK_PER_W // NCHUNK
CGROUPS = CTOK // LANES
GUNROLL = 4


def _gate_block(x0_ref, x1_ref, w_ref, b_ref, out_ref):
    lt0 = lax.dot_general(
        w_ref[...], x0_ref[...],
        dimension_numbers=(((1,), (1,)), ((), ())),
        preferred_element_type=jnp.float32,
    )
    lt1 = lax.dot_general(
        w_ref[...], x1_ref[...],
        dimension_numbers=(((1,), (1,)), ((), ())),
        preferred_element_type=jnp.float32,
    )
    logits_t = jnp.concatenate([lt0, lt1], axis=1) + b_ref[...]
    out_ref[...] = jnp.stack(
        [logits_t[:, j * TOK_PER_W:(j + 1) * TOK_PER_W]
         for j in range(BT // TOK_PER_W)], axis=0)


def _sc_top2(logits_hbm, v1_hbm, v2_hbm, i1_hbm, i2_hbm,
             lg_v, v1_v, v2_v, i1_v, i2_v):
    wid = lax.axis_index("s") * 2 + lax.axis_index("c")
    base = wid * TOK_PER_W
    pltpu.sync_copy(logits_hbm.at[wid], lg_v)

    if True:
        lg = lg_v

        def c_body(gq, carry, lg=lg):
            states = []
            for u in range(GUNROLL):
                m1 = jnp.full((LANES,), -jnp.inf, jnp.float32)
                m2 = jnp.full((LANES,), -jnp.inf, jnp.float32)
                i1 = jnp.zeros((LANES,), jnp.int32)
                i2 = jnp.zeros((LANES,), jnp.int32)
                states.append([m1, m2, i1, i2])
            for e in range(NUM_EXPERTS):
                e_s = jnp.full((LANES,), e, jnp.int32)
                for u in range(GUNROLL):
                    m1, m2, i1, i2 = states[u]
                    v = lg[e, pl.ds((gq * GUNROLL + u) * LANES, LANES)]
                    gt1 = v > m1
                    gt2 = v > m2
                    m2 = jnp.where(gt1, m1, jnp.where(gt2, v, m2))
                    i2 = jnp.where(gt1, i1, jnp.where(gt2, e_s, i2))
                    m1 = jnp.where(gt1, v, m1)
                    i1 = jnp.where(gt1, e_s, i1)
                    states[u] = [m1, m2, i1, i2]
            for u in range(GUNROLL):
                m1, m2, i1, i2 = states[u]
                sl2 = pl.ds((gq * GUNROLL + u) * LANES, LANES)
                v1_v[sl2] = m1
                v2_v[sl2] = m2
                i1_v[sl2] = i1
                i2_v[sl2] = i2
            return carry

        lax.fori_loop(0, (TOK_PER_W // LANES) // GUNROLL, c_body, 0)

    pltpu.sync_copy(v1_v, v1_hbm.at[pl.ds(base, TOK_PER_W)])
    pltpu.sync_copy(v2_v, v2_hbm.at[pl.ds(base, TOK_PER_W)])
    pltpu.sync_copy(i1_v, i1_hbm.at[pl.ds(base, TOK_PER_W)])
    pltpu.sync_copy(i2_v, i2_hbm.at[pl.ds(base, TOK_PER_W)])


@jax.jit
def kernel(x, W, b):
    b_col = b.reshape(NUM_EXPERTS, 1)
    logits = pl.pallas_call(
        _gate_block,
        grid=(N_TOKENS // BT,),
        in_specs=[
            pl.BlockSpec((HBT, DIM_IN), lambda i: (2 * i, 0)),
            pl.BlockSpec((HBT, DIM_IN), lambda i: (2 * i + 1, 0)),
            pl.BlockSpec((NUM_EXPERTS, DIM_IN), lambda i: (0, 0)),
            pl.BlockSpec((NUM_EXPERTS, 1), lambda i: (0, 0)),
        ],
        out_specs=pl.BlockSpec((BT // TOK_PER_W, NUM_EXPERTS, TOK_PER_W),
                               lambda i: (i, 0, 0)),
        out_shape=jax.ShapeDtypeStruct((NUM_WORKERS, NUM_EXPERTS, TOK_PER_W),
                                       jnp.float32),
    )(x, x, W, b_col)

    sc_call = functools.partial(
        pl.kernel,
        mesh=plsc.VectorSubcoreMesh(core_axis_name="c", subcore_axis_name="s"),
        out_type=[
            jax.ShapeDtypeStruct((N_TOKENS,), jnp.float32),
            jax.ShapeDtypeStruct((N_TOKENS,), jnp.float32),
            jax.ShapeDtypeStruct((N_TOKENS,), jnp.int32),
            jax.ShapeDtypeStruct((N_TOKENS,), jnp.int32),
        ],
        scratch_types=[
            pltpu.VMEM((NUM_EXPERTS, TOK_PER_W), jnp.float32),
            pltpu.VMEM((TOK_PER_W,), jnp.float32),
            pltpu.VMEM((TOK_PER_W,), jnp.float32),
            pltpu.VMEM((TOK_PER_W,), jnp.int32),
            pltpu.VMEM((TOK_PER_W,), jnp.int32),
        ],
    )(_sc_top2)
    v1, v2, i1, i2 = sc_call(logits)
    vals = jnp.stack([v1, v2], axis=1)
    idx = jnp.stack([i1, i2], axis=1)
    return (vals, idx)


# speedup vs baseline: 1.0216x; 1.0009x over previous
import functools

import jax
import jax.numpy as jnp
from jax import lax
from jax.experimental import pallas as pl
from jax.experimental.pallas import tpu as pltpu
from jax.experimental.pallas import tpu_sc as plsc

N_TOKENS = 32768
DIM_IN = 4096
NUM_EXPERTS = 64
BT = 1024
HBT = BT // 2

NUM_WORKERS = 32
TOK_PER_W = N_TOKENS // NUM_WORKERS
LANES = 16
GUNROLL = 4


def _gate_block(x0_ref, x1_ref, w_ref, b_ref, out_ref):
    lt0 = lax.dot_general(
        w_ref[...], x0_ref[...],
        dimension_numbers=(((1,), (1,)), ((), ())),
        preferred_element_type=jnp.float32,
    )
    lt1 = lax.dot_general(
        w_ref[...], x1_ref[...],
        dimension_numbers=(((1,), (1,)), ((), ())),
        preferred_element_type=jnp.float32,
    )
    logits_t = jnp.concatenate([lt0, lt1], axis=1) + b_ref[...]
    out_ref[...] = jnp.stack(
        [logits_t[:, j * TOK_PER_W:(j + 1) * TOK_PER_W]
         for j in range(BT // TOK_PER_W)], axis=0)


def _sc_top2(logits_hbm, v1_hbm, v2_hbm, i1_hbm, i2_hbm,
             lg_v, v1_v, v2_v, i1_v, i2_v):
    wid = lax.axis_index("s") * 2 + lax.axis_index("c")
    base = wid * TOK_PER_W
    pltpu.sync_copy(logits_hbm.at[wid], lg_v)

    def g_body(gq, carry):
        states = []
        for u in range(GUNROLL):
            m1 = jnp.full((LANES,), -jnp.inf, jnp.float32)
            m2 = jnp.full((LANES,), -jnp.inf, jnp.float32)
            i1 = jnp.zeros((LANES,), jnp.int32)
            i2 = jnp.zeros((LANES,), jnp.int32)
            states.append([m1, m2, i1, i2])
        for e in range(NUM_EXPERTS):
            e_s = jnp.full((LANES,), e, jnp.int32)
            for u in range(GUNROLL):
                m1, m2, i1, i2 = states[u]
                v = lg_v[e, pl.ds((gq * GUNROLL + u) * LANES, LANES)]
                gt1 = v > m1
                gt2 = v > m2
                m2 = jnp.where(gt1, m1, jnp.where(gt2, v, m2))
                i2 = jnp.where(gt1, i1, jnp.where(gt2, e_s, i2))
                m1 = jnp.where(gt1, v, m1)
                i1 = jnp.where(gt1, e_s, i1)
                states[u] = [m1, m2, i1, i2]
        for u in range(GUNROLL):
            m1, m2, i1, i2 = states[u]
            sl = pl.ds((gq * GUNROLL + u) * LANES, LANES)
            v1_v[sl] = m1
            v2_v[sl] = m2
            i1_v[sl] = i1
            i2_v[sl] = i2
        return carry

    lax.fori_loop(0, (TOK_PER_W // LANES) // GUNROLL, g_body, 0)

    pltpu.sync_copy(v1_v, v1_hbm.at[pl.ds(base, TOK_PER_W)])
    pltpu.sync_copy(v2_v, v2_hbm.at[pl.ds(base, TOK_PER_W)])
    pltpu.sync_copy(i1_v, i1_hbm.at[pl.ds(base, TOK_PER_W)])
    pltpu.sync_copy(i2_v, i2_hbm.at[pl.ds(base, TOK_PER_W)])


@jax.jit
def kernel(x, W, b):
    b_col = b.reshape(NUM_EXPERTS, 1)
    logits = pl.pallas_call(
        _gate_block,
        grid=(N_TOKENS // BT,),
        in_specs=[
            pl.BlockSpec((HBT, DIM_IN), lambda i: (2 * i, 0)),
            pl.BlockSpec((HBT, DIM_IN), lambda i: (2 * i + 1, 0)),
            pl.BlockSpec((NUM_EXPERTS, DIM_IN), lambda i: (0, 0)),
            pl.BlockSpec((NUM_EXPERTS, 1), lambda i: (0, 0)),
        ],
        out_specs=pl.BlockSpec((BT // TOK_PER_W, NUM_EXPERTS, TOK_PER_W),
                               lambda i: (i, 0, 0)),
        out_shape=jax.ShapeDtypeStruct((NUM_WORKERS, NUM_EXPERTS, TOK_PER_W),
                                       jnp.float32),
    )(x, x, W, b_col)

    sc_call = functools.partial(
        pl.kernel,
        mesh=plsc.VectorSubcoreMesh(core_axis_name="c", subcore_axis_name="s"),
        out_type=[
            jax.ShapeDtypeStruct((N_TOKENS,), jnp.float32),
            jax.ShapeDtypeStruct((N_TOKENS,), jnp.float32),
            jax.ShapeDtypeStruct((N_TOKENS,), jnp.int32),
            jax.ShapeDtypeStruct((N_TOKENS,), jnp.int32),
        ],
        scratch_types=[
            pltpu.VMEM((NUM_EXPERTS, TOK_PER_W), jnp.float32),
            pltpu.VMEM((TOK_PER_W,), jnp.float32),
            pltpu.VMEM((TOK_PER_W,), jnp.float32),
            pltpu.VMEM((TOK_PER_W,), jnp.int32),
            pltpu.VMEM((TOK_PER_W,), jnp.int32),
        ],
    )(_sc_top2)
    v1, v2, i1, i2 = sc_call(logits)
    vals = jnp.stack([v1, v2], axis=1)
    idx = jnp.stack([i1, i2], axis=1)
    return (vals, idx)
